# baseline (speedup 1.0000x reference)
import functools

import jax
import jax.numpy as jnp
from jax import lax
from jax.experimental import pallas as pl
from jax.experimental.pallas import tpu as pltpu
from jax.experimental.pallas import tpu_sc as plsc

_VOCAB = 100000
_D = 64
_EPS = 1e-5


_ROWS_PER_BLOCK = 1000


def _table_body(pred_ref, bool_ref, gp_ref, bp_ref, gb_ref, bb_ref, out_ref):
    x = pred_ref[...]
    m = jnp.mean(x, axis=-1, keepdims=True)
    v = jnp.mean((x - m) ** 2, axis=-1, keepdims=True)
    xn = (x - m) / jnp.sqrt(v + _EPS) * gp_ref[...] + bp_ref[...]
    b = bool_ref[...]
    bm = jnp.mean(b, axis=-1, keepdims=True)
    bv = jnp.mean((b - bm) ** 2, axis=-1, keepdims=True)
    bn = (b - bm) / jnp.sqrt(bv + _EPS) * gb_ref[...] + bb_ref[...]
    out_ref[...] = jnp.concatenate([xn * bn[0:1], xn * bn[1:2]], axis=-1)


def _build_combined_table(pred_table, bool_table, gamma_p, beta_p, gamma_b, beta_b):
    r = _ROWS_PER_BLOCK
    return pl.pallas_call(
        _table_body,
        grid=(_VOCAB // r,),
        in_specs=[
            pl.BlockSpec((r, _D), lambda i: (i, 0)),
            pl.BlockSpec((2, _D), lambda i: (0, 0)),
            pl.BlockSpec((1, _D), lambda i: (0, 0)),
            pl.BlockSpec((1, _D), lambda i: (0, 0)),
            pl.BlockSpec((1, _D), lambda i: (0, 0)),
            pl.BlockSpec((1, _D), lambda i: (0, 0)),
        ],
        out_specs=pl.BlockSpec((r, 2 * _D), lambda i: (i, 0)),
        out_shape=jax.ShapeDtypeStruct((_VOCAB, 2 * _D), jnp.float32),
    )(
        pred_table,
        bool_table,
        gamma_p.reshape(1, _D),
        beta_p.reshape(1, _D),
        gamma_b.reshape(1, _D),
        beta_b.reshape(1, _D),
    )


_FG = 8
_SUB = 128


def _make_gather_kernel(b_dim, f_dim, f_pad, nc, ns):
    nw = nc * ns
    b_per_w = b_dim // nw
    n_sub = b_per_w // _SUB
    n_blocks = f_pad // _FG
    n_steps = _FG * n_sub
    mesh = plsc.VectorSubcoreMesh(core_axis_name="c", subcore_axis_name="s")

    @functools.partial(
        pl.kernel,
        out_type=jax.ShapeDtypeStruct((f_dim, _D, b_dim), jnp.float32),
        mesh=mesh,
        compiler_params=pltpu.CompilerParams(needs_layout_passes=False),
        scratch_types=[
            pltpu.VMEM((_FG, b_per_w), jnp.int32),
            pltpu.VMEM((_FG, b_per_w), jnp.int32),
            pltpu.VMEM((2, _SUB, 2 * _D), jnp.float32),
            pltpu.VMEM((2, _D, _SUB), jnp.float32),
            pltpu.SemaphoreType.DMA,
            pltpu.SemaphoreType.DMA,
            pltpu.SemaphoreType.DMA,
            pltpu.SemaphoreType.DMA,
        ],
    )
    def _gather(vvT_hbm, vtT_hbm, comb_hbm, out_hbm,
                vv_v, vt_v, rows_v, tr_v, gsem0, gsem1, osem0, osem1):
        wid = lax.axis_index("s") * nc + lax.axis_index("c")
        b0 = pl.multiple_of(wid * b_per_w, b_per_w)
        lanes = lax.iota(jnp.int32, 16)
        gsems = (gsem0, gsem1)
        osems = (osem0, osem1)

        def fire_gather(s):
            fl, sub = divmod(s, n_sub)
            p = s % 2
            return pltpu.async_copy(
                comb_hbm.at[vt_v.at[fl, pl.ds(sub * _SUB, _SUB)]],
                rows_v.at[p],
                gsems[p],
            )

        def blk_body(blk, carry):
            f0 = pl.multiple_of(blk * _FG, _FG)
            pltpu.sync_copy(vvT_hbm.at[pl.ds(f0, _FG), pl.ds(b0, b_per_w)], vv_v)
            pltpu.sync_copy(vtT_hbm.at[pl.ds(f0, _FG), pl.ds(b0, b_per_w)], vt_v)

            gcopies = {}
            ocopies = {}
            gcopies[0] = fire_gather(0)
            for s in range(n_steps):
                fl, sub = divmod(s, n_sub)
                p = s % 2
                if s + 1 < n_steps:
                    gcopies[s + 1] = fire_gather(s + 1)
                gcopies[s].wait()
                if s - 2 >= 0:
                    ocopies[s - 2].wait()

                rows2d = rows_v.at[p]
                tr2d = tr_v.at[p]

                @plsc.parallel_loop(0, (_SUB // 16) * 16, unroll=8)
                def _tg(i):
                    g = i >> 4
                    k = i & 15
                    bl = lanes + g * 16
                    vv16 = vv_v[fl, pl.ds(sub * _SUB + g * 16, 16)]
                    colb = vv16 * _D
                    rot = (lanes + k) & 15
                    for j in range(_D // 16):
                        dvec = rot + (j * 16)
                        x = plsc.load_gather(rows2d, [bl, colb + dvec])
                        plsc.store_scatter(tr2d, [dvec, bl], x)

                f_eff = lax.min(f0 + fl, f_dim - 1)
                babs = pl.multiple_of(b0 + sub * _SUB, _SUB)
                ocopies[s] = pltpu.async_copy(
                    tr_v.at[p],
                    out_hbm.at[f_eff, :, pl.ds(babs, _SUB)],
                    osems[p],
                )
            ocopies[n_steps - 2].wait()
            ocopies[n_steps - 1].wait()
            return carry

        lax.fori_loop(0, n_blocks, blk_body, 0)

    return _gather


def kernel(var_val, var_type, pred_table, bool_table, gamma_p, beta_p, gamma_b, beta_b):
    b, f = var_val.shape

    comb = _build_combined_table(pred_table, bool_table, gamma_p, beta_p, gamma_b, beta_b)

    info = plsc.get_sparse_core_info()

    f_pad = ((f + _FG - 1) // _FG) * _FG
    vvT = jnp.transpose(var_val.astype(jnp.int32))
    vtT = jnp.transpose(var_type.astype(jnp.int32))
    pad = ((0, f_pad - f), (0, 0))
    vvT = jnp.pad(vvT, pad, mode="edge")
    vtT = jnp.pad(vtT, pad, mode="edge")

    gather = _make_gather_kernel(b, f, f_pad, info.num_cores, info.num_subcores)
    xfd = gather(vvT, vtT, comb)
    return jnp.transpose(xfd, (2, 0, 1))


# speedup vs baseline: 1.0031x; 1.0031x over previous
import functools

import jax
import jax.numpy as jnp
from jax import lax
from jax.experimental import pallas as pl
from jax.experimental.pallas import tpu as pltpu
from jax.experimental.pallas import tpu_sc as plsc

_VOCAB = 100000
_D = 64
_EPS = 1e-5


_ROWS_PER_BLOCK = 1000


def _table_body(pred_ref, bool_ref, gp_ref, bp_ref, gb_ref, bb_ref, out_ref):
    x = pred_ref[...]
    m = jnp.mean(x, axis=-1, keepdims=True)
    v = jnp.mean((x - m) ** 2, axis=-1, keepdims=True)
    xn = (x - m) / jnp.sqrt(v + _EPS) * gp_ref[...] + bp_ref[...]
    b = bool_ref[...]
    bm = jnp.mean(b, axis=-1, keepdims=True)
    bv = jnp.mean((b - bm) ** 2, axis=-1, keepdims=True)
    bn = (b - bm) / jnp.sqrt(bv + _EPS) * gb_ref[...] + bb_ref[...]
    out_ref[...] = jnp.concatenate([xn * bn[0:1], xn * bn[1:2]], axis=-1)


def _build_combined_table(pred_table, bool_table, gamma_p, beta_p, gamma_b, beta_b):
    r = _ROWS_PER_BLOCK
    return pl.pallas_call(
        _table_body,
        grid=(_VOCAB // r,),
        in_specs=[
            pl.BlockSpec((r, _D), lambda i: (i, 0)),
            pl.BlockSpec((2, _D), lambda i: (0, 0)),
            pl.BlockSpec((1, _D), lambda i: (0, 0)),
            pl.BlockSpec((1, _D), lambda i: (0, 0)),
            pl.BlockSpec((1, _D), lambda i: (0, 0)),
            pl.BlockSpec((1, _D), lambda i: (0, 0)),
        ],
        out_specs=pl.BlockSpec((r, 2 * _D), lambda i: (i, 0)),
        out_shape=jax.ShapeDtypeStruct((_VOCAB, 2 * _D), jnp.float32),
    )(
        pred_table,
        bool_table,
        gamma_p.reshape(1, _D),
        beta_p.reshape(1, _D),
        gamma_b.reshape(1, _D),
        beta_b.reshape(1, _D),
    )


_FG = 8
_SUB = 128


def _make_gather_kernel(b_dim, f_dim, f_pad, nc, ns):
    nw = nc * ns
    b_per_w = b_dim // nw
    n_sub = b_per_w // _SUB
    n_blocks = f_pad // _FG
    n_steps = _FG * n_sub
    mesh = plsc.VectorSubcoreMesh(core_axis_name="c", subcore_axis_name="s")

    @functools.partial(
        pl.kernel,
        out_type=jax.ShapeDtypeStruct((f_dim, _D, b_dim), jnp.float32),
        mesh=mesh,
        compiler_params=pltpu.CompilerParams(needs_layout_passes=False),
        scratch_types=[
            pltpu.VMEM((_FG, b_per_w), jnp.int32),
            pltpu.VMEM((_FG, b_per_w), jnp.int32),
            pltpu.VMEM((2, _SUB, 2 * _D), jnp.float32),
            pltpu.VMEM((2, _D, _SUB), jnp.float32),
            pltpu.SemaphoreType.DMA,
            pltpu.SemaphoreType.DMA,
            pltpu.SemaphoreType.DMA,
            pltpu.SemaphoreType.DMA,
        ],
    )
    def _gather(vvT_hbm, vtT_hbm, comb_hbm, out_hbm,
                vv_v, vt_v, rows_v, tr_v, gsem0, gsem1, osem0, osem1):
        wid = lax.axis_index("s") * nc + lax.axis_index("c")
        b0 = pl.multiple_of(wid * b_per_w, b_per_w)
        lanes = lax.iota(jnp.int32, 16)
        gsems = (gsem0, gsem1)
        osems = (osem0, osem1)

        def fire_gather(s):
            fl, sub = divmod(s, n_sub)
            p = s % 2
            return pltpu.async_copy(
                comb_hbm.at[vt_v.at[fl, pl.ds(sub * _SUB, _SUB)]],
                rows_v.at[p],
                gsems[p],
            )

        def blk_body(blk, carry):
            f0 = pl.multiple_of(blk * _FG, _FG)
            pltpu.sync_copy(vvT_hbm.at[pl.ds(f0, _FG), pl.ds(b0, b_per_w)], vv_v)
            pltpu.sync_copy(vtT_hbm.at[pl.ds(f0, _FG), pl.ds(b0, b_per_w)], vt_v)

            gcopies = {}
            ocopies = {}
            gcopies[0] = fire_gather(0)
            for s in range(n_steps):
                fl, sub = divmod(s, n_sub)
                p = s % 2
                if s + 1 < n_steps:
                    gcopies[s + 1] = fire_gather(s + 1)
                gcopies[s].wait()
                if s - 2 >= 0:
                    ocopies[s - 2].wait()

                rows2d = rows_v.at[p]
                tr2d = tr_v.at[p]

                @plsc.parallel_loop(0, (_SUB // 16) * 16, unroll=4)
                def _tg(i):
                    g = i >> 4
                    k = i & 15
                    bl = lanes + g * 16
                    vv16 = vv_v[fl, pl.ds(sub * _SUB + g * 16, 16)]
                    colb = vv16 * _D
                    rot = (lanes + k) & 15
                    for j in range(_D // 16):
                        dvec = rot + (j * 16)
                        x = plsc.load_gather(rows2d, [bl, colb + dvec])
                        plsc.store_scatter(tr2d, [dvec, bl], x)

                f_eff = lax.min(f0 + fl, f_dim - 1)
                babs = pl.multiple_of(b0 + sub * _SUB, _SUB)
                ocopies[s] = pltpu.async_copy(
                    tr_v.at[p],
                    out_hbm.at[f_eff, :, pl.ds(babs, _SUB)],
                    osems[p],
                )
            ocopies[n_steps - 2].wait()
            ocopies[n_steps - 1].wait()
            return carry

        lax.fori_loop(0, n_blocks, blk_body, 0)

    return _gather


def kernel(var_val, var_type, pred_table, bool_table, gamma_p, beta_p, gamma_b, beta_b):
    b, f = var_val.shape

    comb = _build_combined_table(pred_table, bool_table, gamma_p, beta_p, gamma_b, beta_b)

    info = plsc.get_sparse_core_info()

    f_pad = ((f + _FG - 1) // _FG) * _FG
    vvT = jnp.transpose(var_val.astype(jnp.int32))
    vtT = jnp.transpose(var_type.astype(jnp.int32))
    pad = ((0, f_pad - f), (0, 0))
    vvT = jnp.pad(vvT, pad, mode="edge")
    vtT = jnp.pad(vtT, pad, mode="edge")

    gather = _make_gather_kernel(b, f, f_pad, info.num_cores, info.num_subcores)
    xfd = gather(vvT, vtT, comb)
    return jnp.transpose(xfd, (2, 0, 1))


# speedup vs baseline: 1.0594x; 1.0561x over previous
import functools

import jax
import jax.numpy as jnp
from jax import lax
from jax.experimental import pallas as pl
from jax.experimental.pallas import tpu as pltpu
from jax.experimental.pallas import tpu_sc as plsc

_VOCAB = 100000
_D = 64
_EPS = 1e-5


_ROWS_PER_BLOCK = 1024


def _table_body(pred_ref, bool_ref, gp_ref, bp_ref, gb_ref, bb_ref, out_ref):
    x = jnp.transpose(pred_ref[...])
    m = jnp.mean(x, axis=-1, keepdims=True)
    v = jnp.mean((x - m) ** 2, axis=-1, keepdims=True)
    xn = (x - m) / jnp.sqrt(v + _EPS) * gp_ref[...] + bp_ref[...]
    b = bool_ref[...]
    bm = jnp.mean(b, axis=-1, keepdims=True)
    bv = jnp.mean((b - bm) ** 2, axis=-1, keepdims=True)
    bn = (b - bm) / jnp.sqrt(bv + _EPS) * gb_ref[...] + bb_ref[...]
    out_ref[...] = jnp.concatenate([xn * bn[0:1], xn * bn[1:2]], axis=-1)


def _build_combined_table(pred_table, bool_table, gamma_p, beta_p, gamma_b, beta_b):
    r = _ROWS_PER_BLOCK
    return pl.pallas_call(
        _table_body,
        grid=((_VOCAB + r - 1) // r,),
        in_specs=[
            pl.BlockSpec((_D, r), lambda i: (0, i)),
            pl.BlockSpec((2, _D), lambda i: (0, 0)),
            pl.BlockSpec((1, _D), lambda i: (0, 0)),
            pl.BlockSpec((1, _D), lambda i: (0, 0)),
            pl.BlockSpec((1, _D), lambda i: (0, 0)),
            pl.BlockSpec((1, _D), lambda i: (0, 0)),
        ],
        out_specs=pl.BlockSpec((r, 2 * _D), lambda i: (i, 0)),
        out_shape=jax.ShapeDtypeStruct((_VOCAB, 2 * _D), jnp.float32),
    )(
        jnp.transpose(pred_table),
        bool_table,
        gamma_p.reshape(1, _D),
        beta_p.reshape(1, _D),
        gamma_b.reshape(1, _D),
        beta_b.reshape(1, _D),
    )


_FG = 8
_SUB = 128


def _make_gather_kernel(b_dim, f_dim, f_pad, nc, ns):
    nw = nc * ns
    b_per_w = b_dim // nw
    n_sub = b_per_w // _SUB
    n_blocks = f_pad // _FG
    n_steps = _FG * n_sub
    mesh = plsc.VectorSubcoreMesh(core_axis_name="c", subcore_axis_name="s")

    @functools.partial(
        pl.kernel,
        out_type=jax.ShapeDtypeStruct((f_dim, _D, b_dim), jnp.float32),
        mesh=mesh,
        compiler_params=pltpu.CompilerParams(needs_layout_passes=False),
        scratch_types=[
            pltpu.VMEM((_FG, b_per_w), jnp.int32),
            pltpu.VMEM((_FG, b_per_w), jnp.int32),
            pltpu.VMEM((2, _SUB, 2 * _D), jnp.float32),
            pltpu.VMEM((2, _D, _SUB), jnp.float32),
            pltpu.SemaphoreType.DMA,
            pltpu.SemaphoreType.DMA,
            pltpu.SemaphoreType.DMA,
            pltpu.SemaphoreType.DMA,
        ],
    )
    def _gather(vvT_hbm, vtT_hbm, comb_hbm, out_hbm,
                vv_v, vt_v, rows_v, tr_v, gsem0, gsem1, osem0, osem1):
        wid = lax.axis_index("s") * nc + lax.axis_index("c")
        b0 = pl.multiple_of(wid * b_per_w, b_per_w)
        lanes = lax.iota(jnp.int32, 16)
        gsems = (gsem0, gsem1)
        osems = (osem0, osem1)

        def fire_gather(s):
            fl, sub = divmod(s, n_sub)
            p = s % 2
            return pltpu.async_copy(
                comb_hbm.at[vt_v.at[fl, pl.ds(sub * _SUB, _SUB)]],
                rows_v.at[p],
                gsems[p],
            )

        def blk_body(blk, carry):
            f0 = pl.multiple_of(blk * _FG, _FG)
            pltpu.sync_copy(vvT_hbm.at[pl.ds(f0, _FG), pl.ds(b0, b_per_w)], vv_v)
            pltpu.sync_copy(vtT_hbm.at[pl.ds(f0, _FG), pl.ds(b0, b_per_w)], vt_v)

            gcopies = {}
            ocopies = {}
            gcopies[0] = fire_gather(0)
            for s in range(n_steps):
                fl, sub = divmod(s, n_sub)
                p = s % 2
                if s + 1 < n_steps:
                    gcopies[s + 1] = fire_gather(s + 1)
                gcopies[s].wait()
                if s - 2 >= 0:
                    ocopies[s - 2].wait()

                rows2d = rows_v.at[p]
                tr2d = tr_v.at[p]

                @plsc.parallel_loop(0, (_SUB // 16) * 16, unroll=4)
                def _tg(i):
                    g = i >> 4
                    k = i & 15
                    bl = lanes + g * 16
                    vv16 = vv_v[fl, pl.ds(sub * _SUB + g * 16, 16)]
                    colb = vv16 * _D
                    rot = (lanes + k) & 15
                    for j in range(_D // 16):
                        dvec = rot + (j * 16)
                        x = plsc.load_gather(rows2d, [bl, colb + dvec])
                        plsc.store_scatter(tr2d, [dvec, bl], x)

                f_eff = lax.min(f0 + fl, f_dim - 1)
                babs = pl.multiple_of(b0 + sub * _SUB, _SUB)
                ocopies[s] = pltpu.async_copy(
                    tr_v.at[p],
                    out_hbm.at[f_eff, :, pl.ds(babs, _SUB)],
                    osems[p],
                )
            ocopies[n_steps - 2].wait()
            ocopies[n_steps - 1].wait()
            return carry

        lax.fori_loop(0, n_blocks, blk_body, 0)

    return _gather


def kernel(var_val, var_type, pred_table, bool_table, gamma_p, beta_p, gamma_b, beta_b):
    b, f = var_val.shape

    comb = _build_combined_table(pred_table, bool_table, gamma_p, beta_p, gamma_b, beta_b)

    info = plsc.get_sparse_core_info()

    f_pad = ((f + _FG - 1) // _FG) * _FG
    vvT = jnp.transpose(var_val.astype(jnp.int32))
    vtT = jnp.transpose(var_type.astype(jnp.int32))
    pad = ((0, f_pad - f), (0, 0))
    vvT = jnp.pad(vvT, pad, mode="edge")
    vtT = jnp.pad(vtT, pad, mode="edge")

    gather = _make_gather_kernel(b, f, f_pad, info.num_cores, info.num_subcores)
    xfd = gather(vvT, vtT, comb)
    return jnp.transpose(xfd, (2, 0, 1))
